# Initial kernel scaffold; baseline (speedup 1.0000x reference)
import jax
import jax.numpy as jnp
from jax.experimental import pallas as pl


def kernel(rules, global_to_local, emb_weight, bias):
    raise NotImplementedError("write your pallas kernel here")


# speedup vs baseline: 280.0807x; 280.0807x over previous
import functools

import jax
import jax.numpy as jnp
from jax import lax
from jax.experimental import pallas as pl
from jax.experimental.pallas import tpu as pltpu
from jax.experimental.pallas import tpu_sc as plsc

NC = 2
NS = 16
NW = NC * NS
LANES = 16


def _sc_kernel(B, L, W_words, V_pad):
    rows_per_tile = B // NW
    elems = rows_per_tile * L
    groups_per_blk = 8
    n_blocks = rows_per_tile // 16

    mesh = plsc.VectorSubcoreMesh(
        core_axis_name="c", subcore_axis_name="s",
        num_cores=NC, num_subcores=NS)

    @functools.partial(
        pl.kernel,
        out_type=jax.ShapeDtypeStruct((B,), jnp.float32),
        mesh=mesh,
        scratch_types=[
            pltpu.VMEM((W_words,), jnp.int32),
            pltpu.VMEM((V_pad,), jnp.float32),
            pltpu.VMEM((elems,), jnp.int32),
            pltpu.VMEM((rows_per_tile,), jnp.float32),
            pltpu.VMEM((LANES,), jnp.float32),
            pltpu.SemaphoreType.DMA,
        ],
        compiler_params=pltpu.CompilerParams(needs_layout_passes=False),
    )
    def body(g2l_hbm, emb_hbm, rules_hbm, bias_hbm, out_hbm,
             g2l_v, emb_v, rules_v, out_v, bias_v, sem):
        wid = lax.axis_index("s") * NC + lax.axis_index("c")
        base = wid * elems

        c1 = pltpu.async_copy(g2l_hbm, g2l_v, sem)
        c2 = pltpu.async_copy(emb_hbm, emb_v, sem)
        c3 = pltpu.async_copy(rules_hbm.at[pl.ds(base, elems)], rules_v, sem)
        c4 = pltpu.async_copy(bias_hbm, bias_v, sem)
        c1.wait()
        c2.wait()
        c3.wait()
        c4.wait()

        lane = lax.iota(jnp.int32, LANES)
        m_lo = lane < 8
        bias_vec = bias_v[...]

        def blk(i, carry):
            blk_base = i * (16 * L)
            acc = jnp.zeros((LANES,), jnp.float32)
            for g in range(groups_per_blk):
                off = blk_base + g * (2 * L)
                s = jnp.zeros((LANES,), jnp.float32)
                t = jnp.zeros((LANES,), jnp.float32)
                for j in range(25):
                    r = rules_v[pl.ds(off + j * LANES, LANES)]
                    w = plsc.load_gather(g2l_v, [jnp.right_shift(r, 1)])
                    hi = jnp.bitwise_and(jnp.right_shift(w, 16), 0xFFFF)
                    lo = jnp.bitwise_and(w, 0xFFFF)
                    local = jnp.where(jnp.bitwise_and(r, 1) == 1, hi, lo)
                    v = plsc.load_gather(emb_v, [local])
                    sq = v * v
                    if j < 12:
                        s = s + sq
                    elif j == 12:
                        s = s + jnp.where(m_lo, sq, 0.0)
                        t = t + jnp.where(m_lo, 0.0, sq)
                    else:
                        t = t + sq
                r0 = jnp.sum(s)
                r1 = jnp.sum(t)
                acc = jnp.where(lane == 2 * g, r0, acc)
                acc = jnp.where(lane == 2 * g + 1, r1, acc)
            out_v[pl.ds(i * 16, 16)] = acc + bias_vec
            return carry

        lax.fori_loop(0, n_blocks, blk, 0)
        pltpu.sync_copy(out_v, out_hbm.at[pl.ds(wid * rows_per_tile, rows_per_tile)])

    return body


def kernel(rules, global_to_local, emb_weight, bias):
    B, L = rules.shape
    V = emb_weight.shape[0]
    G = global_to_local.shape[0]

    gp = global_to_local.astype(jnp.int32)
    packed = jnp.bitwise_or(gp[0::2], jnp.left_shift(gp[1::2], 16))
    n_words = (G + 1) // 2
    W_words = (n_words + 15) // 16 * 16
    packed = jnp.pad(packed, (0, W_words - n_words))

    V_pad = (V + 15) // 16 * 16
    emb_p = jnp.pad(emb_weight.reshape(-1), (0, V_pad - V))

    bias_vec = jnp.broadcast_to(bias.reshape(()), (LANES,)).astype(jnp.float32)
    rules_flat = rules.reshape(-1).astype(jnp.int32)

    out = _sc_kernel(B, L, W_words, V_pad)(packed, emb_p, rules_flat, bias_vec)
    return out.reshape(B, 1)


# speedup vs baseline: 392.2706x; 1.4006x over previous
import functools

import jax
import jax.numpy as jnp
from jax import lax
from jax.experimental import pallas as pl
from jax.experimental.pallas import tpu as pltpu
from jax.experimental.pallas import tpu_sc as plsc

NC = 2
NS = 16
NW = NC * NS
LANES = 16


def _sc_kernel(B, L, W_words, V_pad, HALF):
    rows_per_tile = B // NW
    elems = rows_per_tile * L
    groups_per_blk = 8
    n_blocks = rows_per_tile // 16

    mesh = plsc.VectorSubcoreMesh(
        core_axis_name="c", subcore_axis_name="s",
        num_cores=NC, num_subcores=NS)

    @functools.partial(
        pl.kernel,
        out_type=jax.ShapeDtypeStruct((B,), jnp.float32),
        mesh=mesh,
        scratch_types=[
            pltpu.VMEM((W_words,), jnp.int32),
            pltpu.VMEM((V_pad,), jnp.float32),
            pltpu.VMEM((elems,), jnp.int32),
            pltpu.VMEM((rows_per_tile,), jnp.float32),
            pltpu.VMEM((LANES,), jnp.float32),
            pltpu.SemaphoreType.DMA,
        ],
        compiler_params=pltpu.CompilerParams(needs_layout_passes=False),
    )
    def body(g2l_hbm, emb_hbm, rules_hbm, bias_hbm, out_hbm,
             g2l_v, emb_v, rules_v, out_v, bias_v, sem):
        wid = lax.axis_index("s") * NC + lax.axis_index("c")
        base = wid * elems

        c1 = pltpu.async_copy(g2l_hbm, g2l_v, sem)
        c2 = pltpu.async_copy(emb_hbm, emb_v, sem)
        c3 = pltpu.async_copy(rules_hbm.at[pl.ds(base, elems)], rules_v, sem)
        c4 = pltpu.async_copy(bias_hbm, bias_v, sem)
        c1.wait()
        c2.wait()
        c3.wait()
        c4.wait()

        lane = lax.iota(jnp.int32, LANES)
        m_lo = lane < 8
        bias_vec = bias_v[...]

        def blk(i, carry):
            blk_base = i * (16 * L)
            acc = jnp.zeros((LANES,), jnp.float32)
            for g in range(groups_per_blk):
                off = blk_base + g * (2 * L)
                s = jnp.zeros((LANES,), jnp.float32)
                t = jnp.zeros((LANES,), jnp.float32)
                for j in range(25):
                    r = rules_v[pl.ds(off + j * LANES, LANES)]
                    in_hi = r >= HALF
                    word_idx = jnp.where(in_hi, r - HALF, r)
                    w = plsc.load_gather(g2l_v, [word_idx])
                    hi = jnp.bitwise_and(jnp.right_shift(w, 16), 0xFFFF)
                    lo = jnp.bitwise_and(w, 0xFFFF)
                    local = jnp.where(in_hi, hi, lo)
                    v = plsc.load_gather(emb_v, [local])
                    sq = v * v
                    if j < 12:
                        s = s + sq
                    elif j == 12:
                        s = s + jnp.where(m_lo, sq, 0.0)
                        t = t + jnp.where(m_lo, 0.0, sq)
                    else:
                        t = t + sq
                r0 = jnp.sum(s)
                r1 = jnp.sum(t)
                acc = jnp.where(lane == 2 * g, r0, acc)
                acc = jnp.where(lane == 2 * g + 1, r1, acc)
            out_v[pl.ds(i * 16, 16)] = acc + bias_vec
            return carry

        lax.fori_loop(0, n_blocks, blk, 0)
        pltpu.sync_copy(out_v, out_hbm.at[pl.ds(wid * rows_per_tile, rows_per_tile)])

    return body


def kernel(rules, global_to_local, emb_weight, bias):
    B, L = rules.shape
    V = emb_weight.shape[0]
    G = global_to_local.shape[0]

    gp = global_to_local.astype(jnp.int32)
    half = (G + 1) // 2
    packed = jnp.bitwise_or(gp[:half], jnp.left_shift(gp[half:2 * half], 16))
    W_words = (half + 15) // 16 * 16
    packed = jnp.pad(packed, (0, W_words - half))

    V_pad = (V + 15) // 16 * 16
    emb_p = jnp.pad(emb_weight.reshape(-1), (0, V_pad - V))

    bias_vec = jnp.broadcast_to(bias.reshape(()), (LANES,)).astype(jnp.float32)
    rules_flat = rules.reshape(-1).astype(jnp.int32)

    out = _sc_kernel(B, L, W_words, V_pad, half)(packed, emb_p, rules_flat, bias_vec)
    return out.reshape(B, 1)
